# baseline (device time: 19256 ns/iter reference)
import jax
import jax.numpy as jnp
from jax import lax
from jax.experimental import pallas as pl
from jax.experimental.pallas import tpu as pltpu

N_CHUNKS = 16


def kernel(x):
    m, n = x.shape
    C = N_CHUNKS
    q = m // C

    def body(x_ref, out_ref, comm_ref, send_sems, recv_sems):
        my_x = lax.axis_index("x")
        my_y = lax.axis_index("y")
        x_nbr = (1 - my_x, my_y)
        y_nbr = (my_x, 1 - my_y)

        barrier = pltpu.get_barrier_semaphore()
        for nbr in (x_nbr, y_nbr):
            pl.semaphore_signal(
                barrier, inc=1, device_id=nbr,
                device_id_type=pl.DeviceIdType.MESH,
            )

        def copy(src_slot, dst_slot, sem, nbr):
            return pltpu.make_async_remote_copy(
                src_ref=comm_ref.at[src_slot],
                dst_ref=comm_ref.at[dst_slot],
                send_sem=send_sems.at[sem],
                recv_sem=recv_sems.at[sem],
                device_id=nbr,
                device_id_type=pl.DeviceIdType.MESH,
            )

        nbr1 = {i: (x_nbr if i < C // 2 else y_nbr) for i in range(C)}
        nbr2 = {i: (y_nbr if i < C // 2 else x_nbr) for i in range(C)}
        order = [j for p in zip(range(C // 2), range(C // 2, C)) for j in p]

        for i in order:
            comm_ref[i] = x_ref[i * q:(i + 1) * q, :].astype(jnp.bfloat16)

        pl.semaphore_wait(barrier, 2)

        r1, r2 = {}, {}
        for i in order:
            r1[i] = copy(i, C + i, i, nbr1[i])
            r1[i].start()

        for i in order:
            r1[i].wait_recv()
            comm_ref[2 * C + i] = comm_ref[i] + comm_ref[C + i]
            r2[i] = copy(2 * C + i, 3 * C + i, C + i, nbr2[i])
            r2[i].start()

        for i in order:
            r2[i].wait_recv()
            out_ref[i * q:(i + 1) * q, :] = comm_ref[2 * C + i] + comm_ref[3 * C + i]

        for i in order:
            r1[i].wait_send()
            r2[i].wait_send()

    return pl.pallas_call(
        body,
        out_shape=jax.ShapeDtypeStruct((m, n), jnp.bfloat16),
        in_specs=[pl.BlockSpec(memory_space=pltpu.VMEM)],
        out_specs=pl.BlockSpec(memory_space=pltpu.VMEM),
        scratch_shapes=[
            pltpu.VMEM((4 * C, q, n), jnp.bfloat16),
            pltpu.SemaphoreType.DMA((2 * C,)),
            pltpu.SemaphoreType.DMA((2 * C,)),
        ],
        compiler_params=pltpu.CompilerParams(collective_id=0),
    )(x)


# device time: 16287 ns/iter; 1.1823x vs baseline; 1.1823x over previous
import jax
import jax.numpy as jnp
from jax import lax
from jax.experimental import pallas as pl
from jax.experimental.pallas import tpu as pltpu

N_CHUNKS = 8


def kernel(x):
    m, n = x.shape
    C = N_CHUNKS
    q = m // C
    hq = q // 2

    def body(x_ref, out_ref, in_ref, half_ref, send_sems, recv_sems):
        my_x = lax.axis_index("x")
        my_y = lax.axis_index("y")
        x_nbr = (1 - my_x, my_y)
        y_nbr = (my_x, 1 - my_y)

        barrier = pltpu.get_barrier_semaphore()
        for nbr in (x_nbr, y_nbr):
            pl.semaphore_signal(
                barrier, inc=1, device_id=nbr,
                device_id_type=pl.DeviceIdType.MESH,
            )

        first = {i: (x_nbr if i < C // 2 else y_nbr) for i in range(C)}
        second = {i: (y_nbr if i < C // 2 else x_nbr) for i in range(C)}
        own = {i: (my_x if i < C // 2 else my_y) for i in range(C)}
        order = [j for p in zip(range(C // 2), range(C // 2, C)) for j in p]

        for i in order:
            in_ref[i] = x_ref[i * q:(i + 1) * q, :].astype(jnp.bfloat16)

        pl.semaphore_wait(barrier, 2)

        def copy(src_ref, dst_ref, sem, nbr):
            return pltpu.make_async_remote_copy(
                src_ref=src_ref,
                dst_ref=dst_ref,
                send_sem=send_sems.at[sem],
                recv_sem=recv_sems.at[sem],
                device_id=nbr,
                device_id_type=pl.DeviceIdType.MESH,
            )

        r1, r2, r3 = {}, {}, {}
        for i in order:
            o = own[i]
            r1[i] = copy(
                in_ref.at[i, pl.ds((1 - o) * hq, hq), :],
                half_ref.at[i],
                i, first[i],
            )
            r1[i].start()

        for i in order:
            o = own[i]
            r1[i].wait_recv()
            half_ref[C + i] = in_ref[i, pl.ds(o * hq, hq), :] + half_ref[i]
            r2[i] = copy(
                half_ref.at[C + i], half_ref.at[2 * C + i], C + i, second[i]
            )
            r2[i].start()

        for i in order:
            r2[i].wait_recv()
            half_ref[3 * C + i] = half_ref[C + i] + half_ref[2 * C + i]
            r3[i] = copy(
                half_ref.at[3 * C + i], half_ref.at[4 * C + i], 2 * C + i,
                first[i],
            )
            r3[i].start()

        for i in order:
            o = own[i]
            r3[i].wait_recv()
            out_ref[pl.ds(i * q + o * hq, hq), :] = half_ref[3 * C + i]
            out_ref[pl.ds(i * q + (1 - o) * hq, hq), :] = half_ref[4 * C + i]

        for i in order:
            r1[i].wait_send()
            r2[i].wait_send()
            r3[i].wait_send()

    return pl.pallas_call(
        body,
        out_shape=jax.ShapeDtypeStruct((m, n), jnp.bfloat16),
        in_specs=[pl.BlockSpec(memory_space=pltpu.VMEM)],
        out_specs=pl.BlockSpec(memory_space=pltpu.VMEM),
        scratch_shapes=[
            pltpu.VMEM((C, q, n), jnp.bfloat16),
            pltpu.VMEM((5 * C, hq, n), jnp.bfloat16),
            pltpu.SemaphoreType.DMA((3 * C,)),
            pltpu.SemaphoreType.DMA((3 * C,)),
        ],
        compiler_params=pltpu.CompilerParams(collective_id=0),
    )(x)
